# Initial kernel scaffold; baseline (speedup 1.0000x reference)
import jax
import jax.numpy as jnp
from jax.experimental import pallas as pl


def kernel(x, edge_index, batch, W1, b1, p1, W2, b2, p2, W3, b3, p3, Wl, bl):
    raise NotImplementedError("write your pallas kernel here")


# speedup vs baseline: 1.0831x; 1.0831x over previous
import functools
import jax
import jax.numpy as jnp
from jax.experimental import pallas as pl

N_BLK = 1024
S_GRAPHS = 64


def _linear_body(x_ref, wt_ref, b_ref, o_ref, *, relu):
    y = jnp.dot(x_ref[...], wt_ref[...], preferred_element_type=jnp.float32)
    y = y + b_ref[0, :][None, :]
    if relu:
        y = jnp.maximum(y, 0.0)
    o_ref[...] = y


def _linear(x, W, b, relu):
    N, K = x.shape
    F = W.shape[0]
    n_pad = (-N) % N_BLK
    xp = jnp.pad(x, ((0, n_pad), (0, 0)))
    Np = N + n_pad
    wt = W.T
    b8 = jnp.broadcast_to(b[None, :], (8, F))
    out = pl.pallas_call(
        functools.partial(_linear_body, relu=relu),
        grid=(Np // N_BLK,),
        in_specs=[
            pl.BlockSpec((N_BLK, K), lambda i: (i, 0)),
            pl.BlockSpec((K, F), lambda i: (0, 0)),
            pl.BlockSpec((8, F), lambda i: (0, 0)),
        ],
        out_specs=pl.BlockSpec((N_BLK, F), lambda i: (i, 0)),
        out_shape=jax.ShapeDtypeStruct((Np, F), jnp.float32),
    )(xp, wt, b8)
    return out[:N]


def _propagate(xin, row, col, w, N):
    deg = jax.ops.segment_sum(w, col, num_segments=N) + 1.0
    dis = jax.lax.rsqrt(deg)
    norm = dis[row] * w * dis[col]
    agg = jax.ops.segment_sum(norm[:, None] * xin[row], col, num_segments=N)
    return agg + (dis * dis)[:, None] * xin


def _topk(x, row, col, w, p, batch_i, valid, N):
    score = jnp.tanh((x @ p) / jnp.linalg.norm(p))
    batch_f = batch_i.astype(x.dtype)
    sort_key = jnp.where(valid > 0, batch_f * 4.0 - score, 4.0 * S_GRAPHS)
    order = jnp.argsort(sort_key)
    n = jax.ops.segment_sum(valid.astype(jnp.int32), batch_i, num_segments=S_GRAPHS)
    k = (3 * n + 4) // 5
    cstart = jnp.concatenate([jnp.zeros((1,), n.dtype), jnp.cumsum(n)[:-1]])
    bs = batch_i[order]
    vs = valid[order]
    rank = jnp.arange(N, dtype=n.dtype) - cstart[bs]
    keep = (vs > 0) & (rank < k[bs])
    keepf = keep.astype(x.dtype)
    x_new = x[order] * score[order][:, None] * keepf[:, None]
    inv = jnp.zeros((N,), dtype=row.dtype).at[order].set(jnp.arange(N, dtype=row.dtype))
    kept = jnp.zeros((N,), dtype=x.dtype).at[order].set(keepf)
    w_new = w * kept[row] * kept[col]
    batch_new = jnp.where(keep, bs, 0)
    return x_new, inv[row], inv[col], w_new, batch_new, keepf, k


def kernel(x, edge_index, batch, W1, b1, p1, W2, b2, p2, W3, b3, p3, Wl, bl):
    N0 = x.shape[0]
    row, col = edge_index[0], edge_index[1]
    w = jnp.ones((row.shape[0],), dtype=x.dtype)
    batch_i = batch.astype(jnp.int32)
    valid = jnp.ones((N0,), dtype=x.dtype)

    h = _linear(_propagate(x, row, col, w, N0), W1, b1, relu=True)
    h, row, col, w, batch_i, valid, k0 = _topk(h, row, col, w, p1, batch_i, valid, N0)

    xl = _linear(h, W2, jnp.zeros_like(b2), relu=False)
    h = jax.nn.relu(_propagate(xl, row, col, w, N0) + b2)
    h, row, col, w, batch_i, valid, k1 = _topk(h, row, col, w, p2, batch_i, valid, N0)

    xl = _linear(h, W3, jnp.zeros_like(b3), relu=False)
    h = jax.nn.relu(_propagate(xl, row, col, w, N0) + b3)
    h, row, col, w, batch_i, valid, k2 = _topk(h, row, col, w, p3, batch_i, valid, N0)

    cnt3 = jnp.maximum(k2, 1).astype(x.dtype)
    x_mean = jax.ops.segment_sum(h, batch_i, num_segments=S_GRAPHS) / cnt3[:, None]
    hm = jnp.where(valid[:, None] > 0, h, -jnp.inf)
    x_max = jax.ops.segment_max(hm, batch_i, num_segments=S_GRAPHS)
    x_max = jnp.where(k2[:, None] > 0, x_max, 0.0)
    feat = jnp.concatenate([x_mean, x_max], axis=1)

    Fin = feat.shape[1]
    wlp = jnp.zeros((Fin, 128), jnp.float32).at[:, 0].set(Wl[0])
    blp = jnp.zeros((128,), jnp.float32).at[0].set(bl[0])
    out = pl.pallas_call(
        functools.partial(_linear_body, relu=False),
        in_specs=[
            pl.BlockSpec((S_GRAPHS, Fin), lambda: (0, 0)),
            pl.BlockSpec((Fin, 128), lambda: (0, 0)),
            pl.BlockSpec((8, 128), lambda: (0, 0)),
        ],
        out_specs=pl.BlockSpec((S_GRAPHS, 128), lambda: (0, 0)),
        out_shape=jax.ShapeDtypeStruct((S_GRAPHS, 128), jnp.float32),
    )(feat, wlp, jnp.broadcast_to(blp[None, :], (8, 128)))
    return out[:, :1]


# speedup vs baseline: 4.1161x; 3.8002x over previous
import functools
import jax
import jax.numpy as jnp
from jax.experimental import pallas as pl

N_BLK = 1024
S_GRAPHS = 64


def _linear_body(x_ref, wt_ref, b_ref, o_ref, *, relu):
    y = jnp.dot(x_ref[...], wt_ref[...], preferred_element_type=jnp.float32)
    y = y + b_ref[0, :][None, :]
    if relu:
        y = jnp.maximum(y, 0.0)
    o_ref[...] = y


def _linear(x, W, b, relu):
    N, K = x.shape
    F = W.shape[0]
    n_pad = (-N) % N_BLK
    xp = jnp.pad(x, ((0, n_pad), (0, 0)))
    Np = N + n_pad
    wt = W.T
    b8 = jnp.broadcast_to(b[None, :], (8, F))
    out = pl.pallas_call(
        functools.partial(_linear_body, relu=relu),
        grid=(Np // N_BLK,),
        in_specs=[
            pl.BlockSpec((N_BLK, K), lambda i: (i, 0)),
            pl.BlockSpec((K, F), lambda i: (0, 0)),
            pl.BlockSpec((8, F), lambda i: (0, 0)),
        ],
        out_specs=pl.BlockSpec((N_BLK, F), lambda i: (i, 0)),
        out_shape=jax.ShapeDtypeStruct((Np, F), jnp.float32),
    )(xp, wt, b8)
    return out[:N]


def _propagate(x, row, col, valid, N):
    if valid is None:
        vr_sum = jax.ops.segment_sum(jnp.ones(row.shape, x.dtype), col,
                                     num_segments=N)
        deg = 1.0 + vr_sum
        dis = jax.lax.rsqrt(deg)
        u = dis
    else:
        vr_sum = jax.ops.segment_sum(valid[row], col, num_segments=N)
        deg = 1.0 + valid * vr_sum
        dis = jax.lax.rsqrt(deg)
        u = dis * valid
    xs = dis[:, None] * x
    agg = jax.ops.segment_sum(xs[row], col, num_segments=N)
    return u[:, None] * agg + (dis * dis)[:, None] * x


def _topk(x, p, batch_i, valid, pos, N):
    score = jnp.tanh((x @ p) / jnp.linalg.norm(p))
    vmask = jnp.ones((N,), x.dtype) if valid is None else valid
    batch_f = batch_i.astype(x.dtype)
    sort_key = jnp.where(vmask > 0, batch_f * 4.0 - score, 4.0 * S_GRAPHS)
    order = jnp.lexsort((pos, sort_key))
    n = jax.ops.segment_sum(vmask.astype(jnp.int32), batch_i,
                            num_segments=S_GRAPHS)
    k = (3 * n + 4) // 5
    cstart = jnp.concatenate([jnp.zeros((1,), n.dtype), jnp.cumsum(n)[:-1]])
    bs = batch_i[order]
    vs = vmask[order]
    rank = jnp.arange(N, dtype=n.dtype) - cstart[bs]
    keep = (vs > 0) & (rank < k[bs])
    iota = jnp.arange(N, dtype=pos.dtype)
    keep_orig = jnp.zeros((N,), x.dtype).at[order].set(keep.astype(x.dtype))
    pos_new = jnp.zeros((N,), pos.dtype).at[order].set(iota)
    x_new = x * score[:, None] * keep_orig[:, None]
    return x_new, keep_orig, pos_new, k


def kernel(x, edge_index, batch, W1, b1, p1, W2, b2, p2, W3, b3, p3, Wl, bl):
    N0 = x.shape[0]
    row, col = edge_index[0], edge_index[1]
    batch_i = batch.astype(jnp.int32)

    pos = jnp.arange(N0, dtype=jnp.int32)
    h = _linear(_propagate(x, row, col, None, N0), W1, b1, relu=True)
    h, valid, pos, k0 = _topk(h, p1, batch_i, None, pos, N0)

    xl = _linear(h, W2, jnp.zeros_like(b2), relu=False)
    h = jax.nn.relu(_propagate(xl, row, col, valid, N0) + b2)
    h, valid, pos, k1 = _topk(h, p2, batch_i, valid, pos, N0)

    xl = _linear(h, W3, jnp.zeros_like(b3), relu=False)
    h = jax.nn.relu(_propagate(xl, row, col, valid, N0) + b3)
    h, valid, pos, k2 = _topk(h, p3, batch_i, valid, pos, N0)

    cnt3 = jnp.maximum(k2, 1).astype(x.dtype)
    x_mean = jax.ops.segment_sum(h, batch_i, num_segments=S_GRAPHS) / cnt3[:, None]
    hm = jnp.where(valid[:, None] > 0, h, -jnp.inf)
    x_max = jax.ops.segment_max(hm, batch_i, num_segments=S_GRAPHS)
    x_max = jnp.where(k2[:, None] > 0, x_max, 0.0)
    feat = jnp.concatenate([x_mean, x_max], axis=1)

    Fin = feat.shape[1]
    wlp = jnp.zeros((Fin, 128), jnp.float32).at[:, 0].set(Wl[0])
    blp = jnp.zeros((128,), jnp.float32).at[0].set(bl[0])
    out = pl.pallas_call(
        functools.partial(_linear_body, relu=False),
        in_specs=[
            pl.BlockSpec((S_GRAPHS, Fin), lambda: (0, 0)),
            pl.BlockSpec((Fin, 128), lambda: (0, 0)),
            pl.BlockSpec((8, 128), lambda: (0, 0)),
        ],
        out_specs=pl.BlockSpec((S_GRAPHS, 128), lambda: (0, 0)),
        out_shape=jax.ShapeDtypeStruct((S_GRAPHS, 128), jnp.float32),
    )(feat, wlp, jnp.broadcast_to(blp[None, :], (8, 128)))
    return out[:, :1]


# speedup vs baseline: 4.7897x; 1.1637x over previous
import functools
import jax
import jax.numpy as jnp
from jax.experimental import pallas as pl

N_BLK = 1024
S_GRAPHS = 64


def _linear_body(x_ref, wt_ref, b_ref, o_ref, *, relu):
    y = jnp.dot(x_ref[...], wt_ref[...], preferred_element_type=jnp.float32)
    y = y + b_ref[0, :][None, :]
    if relu:
        y = jnp.maximum(y, 0.0)
    o_ref[...] = y


def _linear(x, W, b, relu):
    N, K = x.shape
    F = W.shape[0]
    n_pad = (-N) % N_BLK
    xp = jnp.pad(x, ((0, n_pad), (0, 0)))
    Np = N + n_pad
    wt = W.T
    b8 = jnp.broadcast_to(b[None, :], (8, F))
    out = pl.pallas_call(
        functools.partial(_linear_body, relu=relu),
        grid=(Np // N_BLK,),
        in_specs=[
            pl.BlockSpec((N_BLK, K), lambda i: (i, 0)),
            pl.BlockSpec((K, F), lambda i: (0, 0)),
            pl.BlockSpec((8, F), lambda i: (0, 0)),
        ],
        out_specs=pl.BlockSpec((N_BLK, F), lambda i: (i, 0)),
        out_shape=jax.ShapeDtypeStruct((Np, F), jnp.float32),
    )(xp, wt, b8)
    return out[:N]


def _propagate(x, row, col, valid, N):
    if valid is None:
        vr_sum = jax.ops.segment_sum(jnp.ones(row.shape, x.dtype), col,
                                     num_segments=N)
        deg = 1.0 + vr_sum
        dis = jax.lax.rsqrt(deg)
        u = dis
    else:
        v8 = jnp.broadcast_to(valid[:, None], (N, 8))
        vr_sum = jax.ops.segment_sum(v8[row], col, num_segments=N)[:, 0]
        deg = 1.0 + valid * vr_sum
        dis = jax.lax.rsqrt(deg)
        u = dis * valid
    xs = dis[:, None] * x
    agg = jax.ops.segment_sum(xs[row], col, num_segments=N)
    return u[:, None] * agg + (dis * dis)[:, None] * x


def _topk(x, p, batch_i, valid, pos, N):
    score = jnp.tanh((x @ p) / jnp.linalg.norm(p))
    vmask = jnp.ones((N,), x.dtype) if valid is None else valid
    batch_f = batch_i.astype(x.dtype)
    sort_key = jnp.where(vmask > 0, batch_f * 4.0 - score, 4.0 * S_GRAPHS)
    order = jnp.lexsort((pos, sort_key))
    n = jax.ops.segment_sum(vmask.astype(jnp.int32), batch_i,
                            num_segments=S_GRAPHS)
    k = (3 * n + 4) // 5
    cstart = jnp.concatenate([jnp.zeros((1,), n.dtype), jnp.cumsum(n)[:-1]])
    bs = batch_i[order]
    vs = vmask[order]
    rank = jnp.arange(N, dtype=n.dtype) - cstart[bs]
    keep = (vs > 0) & (rank < k[bs])
    iota = jnp.arange(N, dtype=pos.dtype)
    keep_orig = jnp.zeros((N,), x.dtype).at[order].set(keep.astype(x.dtype))
    pos_new = jnp.zeros((N,), pos.dtype).at[order].set(iota)
    x_new = x * score[:, None] * keep_orig[:, None]
    return x_new, keep_orig, pos_new, k


def kernel(x, edge_index, batch, W1, b1, p1, W2, b2, p2, W3, b3, p3, Wl, bl):
    N0 = x.shape[0]
    row, col = edge_index[0], edge_index[1]
    batch_i = batch.astype(jnp.int32)

    pos = jnp.arange(N0, dtype=jnp.int32)
    h = _linear(_propagate(x, row, col, None, N0), W1, b1, relu=True)
    h, valid, pos, k0 = _topk(h, p1, batch_i, None, pos, N0)

    xl = _linear(h, W2, jnp.zeros_like(b2), relu=False)
    h = jax.nn.relu(_propagate(xl, row, col, valid, N0) + b2)
    h, valid, pos, k1 = _topk(h, p2, batch_i, valid, pos, N0)

    xl = _linear(h, W3, jnp.zeros_like(b3), relu=False)
    h = jax.nn.relu(_propagate(xl, row, col, valid, N0) + b3)
    h, valid, pos, k2 = _topk(h, p3, batch_i, valid, pos, N0)

    cnt3 = jnp.maximum(k2, 1).astype(x.dtype)
    x_mean = jax.ops.segment_sum(h, batch_i, num_segments=S_GRAPHS) / cnt3[:, None]
    hm = jnp.where(valid[:, None] > 0, h, -jnp.inf)
    x_max = jax.ops.segment_max(hm, batch_i, num_segments=S_GRAPHS)
    x_max = jnp.where(k2[:, None] > 0, x_max, 0.0)
    feat = jnp.concatenate([x_mean, x_max], axis=1)

    Fin = feat.shape[1]
    wlp = jnp.zeros((Fin, 128), jnp.float32).at[:, 0].set(Wl[0])
    blp = jnp.zeros((128,), jnp.float32).at[0].set(bl[0])
    out = pl.pallas_call(
        functools.partial(_linear_body, relu=False),
        in_specs=[
            pl.BlockSpec((S_GRAPHS, Fin), lambda: (0, 0)),
            pl.BlockSpec((Fin, 128), lambda: (0, 0)),
            pl.BlockSpec((8, 128), lambda: (0, 0)),
        ],
        out_specs=pl.BlockSpec((S_GRAPHS, 128), lambda: (0, 0)),
        out_shape=jax.ShapeDtypeStruct((S_GRAPHS, 128), jnp.float32),
    )(feat, wlp, jnp.broadcast_to(blp[None, :], (8, 128)))
    return out[:, :1]
